# baseline (device time: 105442 ns/iter reference)
import functools

import jax
import jax.numpy as jnp
from jax import lax
from jax.experimental import pallas as pl
from jax.experimental.pallas import tpu as pltpu

N_DEV = 8
B, SQ, SKV, DH = 2, 256, 256, 64
H_PER = 4
HID = H_PER * DH
D_MODEL = 512


def kernel(x, Wq, K_ext, V_ext, Wo):
    pos = lax.axis_index("i")
    wq_my = lax.dynamic_slice_in_dim(Wq, pos * HID, HID, axis=1)
    wo_my = lax.dynamic_slice_in_dim(Wo, pos * HID, HID, axis=0)

    def body(x_ref, wq_ref, k_ref, v_ref, wo_ref, out_ref,
             comm_ref, send_sems, recv_sems):
        my = lax.axis_index("i")
        left = lax.rem(my - 1 + N_DEV, N_DEV)
        right = lax.rem(my + 1, N_DEV)

        barrier_sem = pltpu.get_barrier_semaphore()
        for nbr in (left, right):
            pl.semaphore_signal(barrier_sem, inc=1, device_id=(nbr,),
                                device_id_type=pl.DeviceIdType.MESH)
        pl.semaphore_wait(barrier_sem, 2)

        qi = lax.broadcasted_iota(jnp.int32, (SQ, SKV), 0)
        ki = lax.broadcasted_iota(jnp.int32, (SQ, SKV), 1)
        mask = (jnp.abs(qi - ki) <= 128) | (ki < 32) | (qi < 32)
        for b in range(B):
            q_b = jnp.dot(x_ref[b], wq_ref[...],
                          preferred_element_type=jnp.float32)
            ctx_cols = []
            for h in range(H_PER):
                q_h = q_b[:, h * DH:(h + 1) * DH]
                k_h = k_ref[b, :, h, :]
                v_h = v_ref[b, :, h, :]
                s = lax.dot_general(q_h, k_h, (((1,), (1,)), ((), ())),
                                    preferred_element_type=jnp.float32)
                s = s * 0.125
                s = jnp.where(mask, s, -1e9)
                s = s - jnp.max(s, axis=1, keepdims=True)
                w = jnp.exp(s)
                w = w / jnp.sum(w, axis=1, keepdims=True)
                ctx_cols.append(jnp.dot(w, v_h,
                                        preferred_element_type=jnp.float32))
            ctx_b = jnp.concatenate(ctx_cols, axis=1)
            part_b = jnp.dot(ctx_b, wo_ref[...],
                             preferred_element_type=jnp.float32)
            out_ref[b] = part_b
            comm_ref[0, b] = part_b

        for hop in range(N_DEV - 1):
            rdma = pltpu.make_async_remote_copy(
                src_ref=comm_ref.at[hop],
                dst_ref=comm_ref.at[hop + 1],
                send_sem=send_sems.at[hop],
                recv_sem=recv_sems.at[hop],
                device_id=(right,),
                device_id_type=pl.DeviceIdType.MESH,
            )
            rdma.start()
            rdma.wait()
            out_ref[...] = out_ref[...] + comm_ref[hop + 1]

        @functools.partial(pl.run_scoped, sem=pltpu.SemaphoreType.REGULAR)
        def _(sem):
            for nbr in (left, right):
                pl.semaphore_signal(sem, inc=1, device_id=(nbr,),
                                    device_id_type=pl.DeviceIdType.MESH)
            pl.semaphore_wait(sem, 2)

    return pl.pallas_call(
        body,
        out_shape=jax.ShapeDtypeStruct((B, SQ, D_MODEL), jnp.float32),
        in_specs=[pl.BlockSpec(memory_space=pltpu.VMEM)] * 5,
        out_specs=pl.BlockSpec(memory_space=pltpu.VMEM),
        scratch_shapes=[
            pltpu.VMEM((N_DEV, B, SQ, D_MODEL), jnp.float32),
            pltpu.SemaphoreType.DMA((N_DEV - 1,)),
            pltpu.SemaphoreType.DMA((N_DEV - 1,)),
        ],
        compiler_params=pltpu.CompilerParams(collective_id=0),
    )(x, wq_my, K_ext, V_ext, wo_my)


# device time: 29588 ns/iter; 3.5637x vs baseline; 3.5637x over previous
import jax
import jax.numpy as jnp
from jax import lax
from jax.experimental import pallas as pl
from jax.experimental.pallas import tpu as pltpu

N_DEV = 8
B, SQ, SKV, DH = 2, 256, 256, 64
H_PER = 4
HID = H_PER * DH
D_MODEL = 512
ROWS = B * SQ
CH = ROWS // N_DEV


def kernel(x, Wq, K_ext, V_ext, Wo):
    pos = lax.axis_index("i")
    wq_my = lax.dynamic_slice_in_dim(Wq, pos * HID, HID, axis=1)
    wo_my = lax.dynamic_slice_in_dim(Wo, pos * HID, HID, axis=0)

    def body(x_ref, wq_ref, k_ref, v_ref, wo_ref, out_ref,
             part_ref, rs_buf, chunk_ref, acc_ref,
             send_p1, recv_p1, send_p2, recv_p2):
        my = lax.axis_index("i")

        barrier_sem = pltpu.get_barrier_semaphore()
        for d in range(1, N_DEV):
            pl.semaphore_signal(barrier_sem, inc=1,
                                device_id=(lax.rem(my + d, N_DEV),),
                                device_id_type=pl.DeviceIdType.MESH)
        pl.semaphore_wait(barrier_sem, N_DEV - 1)

        qi = lax.broadcasted_iota(jnp.int32, (SQ, SKV), 0)
        ki = lax.broadcasted_iota(jnp.int32, (SQ, SKV), 1)
        mask = (jnp.abs(qi - ki) <= 128) | (ki < 32) | (qi < 32)
        for b in range(B):
            q_b = jnp.dot(x_ref[b], wq_ref[...],
                          preferred_element_type=jnp.float32)
            ctx_cols = []
            for h in range(H_PER):
                q_h = q_b[:, h * DH:(h + 1) * DH]
                k_h = k_ref[b, :, h, :]
                v_h = v_ref[b, :, h, :]
                s = lax.dot_general(q_h, k_h, (((1,), (1,)), ((), ())),
                                    preferred_element_type=jnp.float32)
                s = s * 0.125
                s = jnp.where(mask, s, -1e9)
                s = s - jnp.max(s, axis=1, keepdims=True)
                w = jnp.exp(s)
                w = w / jnp.sum(w, axis=1, keepdims=True)
                ctx_cols.append(jnp.dot(w, v_h,
                                        preferred_element_type=jnp.float32))
            ctx_b = jnp.concatenate(ctx_cols, axis=1)
            part_ref[b * SQ:(b + 1) * SQ] = jnp.dot(
                ctx_b, wo_ref[...], preferred_element_type=jnp.float32)

        p1 = []
        for d in range(1, N_DEV):
            t = lax.rem(my + d, N_DEV)
            rdma = pltpu.make_async_remote_copy(
                src_ref=part_ref.at[pl.ds(t * CH, CH)],
                dst_ref=rs_buf.at[d],
                send_sem=send_p1.at[d],
                recv_sem=recv_p1.at[d],
                device_id=(t,),
                device_id_type=pl.DeviceIdType.MESH,
            )
            rdma.start()
            p1.append(rdma)

        red = part_ref[pl.ds(my * CH, CH)]
        for d in range(1, N_DEV):
            p1[d - 1].wait_recv()
            red = red + rs_buf[d]
        chunk_ref[...] = red

        p2 = []
        for d in range(1, N_DEV):
            t = lax.rem(my + d, N_DEV)
            rdma = pltpu.make_async_remote_copy(
                src_ref=chunk_ref,
                dst_ref=acc_ref.at[pl.ds(my * CH, CH)],
                send_sem=send_p2.at[d],
                recv_sem=recv_p2.at[d],
                device_id=(t,),
                device_id_type=pl.DeviceIdType.MESH,
            )
            rdma.start()
            p2.append(rdma)
        acc_ref[pl.ds(my * CH, CH)] = red

        for d in range(1, N_DEV):
            p2[d - 1].wait_recv()
        for b in range(B):
            out_ref[b] = acc_ref[b * SQ:(b + 1) * SQ]

        for d in range(1, N_DEV):
            p1[d - 1].wait_send()
            p2[d - 1].wait_send()

    return pl.pallas_call(
        body,
        out_shape=jax.ShapeDtypeStruct((B, SQ, D_MODEL), jnp.float32),
        in_specs=[pl.BlockSpec(memory_space=pltpu.VMEM)] * 5,
        out_specs=pl.BlockSpec(memory_space=pltpu.VMEM),
        scratch_shapes=[
            pltpu.VMEM((ROWS, D_MODEL), jnp.float32),
            pltpu.VMEM((N_DEV, CH, D_MODEL), jnp.float32),
            pltpu.VMEM((CH, D_MODEL), jnp.float32),
            pltpu.VMEM((ROWS, D_MODEL), jnp.float32),
            pltpu.SemaphoreType.DMA((N_DEV,)),
            pltpu.SemaphoreType.DMA((N_DEV,)),
            pltpu.SemaphoreType.DMA((N_DEV,)),
            pltpu.SemaphoreType.DMA((N_DEV,)),
        ],
        compiler_params=pltpu.CompilerParams(collective_id=0),
    )(x, wq_my, K_ext, V_ext, wo_my)


# device time: 23268 ns/iter; 4.5316x vs baseline; 1.2716x over previous
import jax
import jax.numpy as jnp
from jax import lax
from jax.experimental import pallas as pl
from jax.experimental.pallas import tpu as pltpu

N_DEV = 8
B, SQ, SKV, DH = 2, 256, 256, 64
H_PER = 4
HID = H_PER * DH
D_MODEL = 512
ROWS = B * SQ
CH = ROWS // N_DEV


def kernel(x, Wq, K_ext, V_ext, Wo):
    pos = lax.axis_index("i")
    wq_my = lax.dynamic_slice_in_dim(Wq, pos * HID, HID, axis=1)
    wo_my = lax.dynamic_slice_in_dim(Wo, pos * HID, HID, axis=0)

    def body(x_ref, wq_ref, k_ref, v_ref, wo_ref, out_ref,
             part_ref, rs_buf, chunk_ref, acc_ref,
             send_p1, recv_p1, send_p2, recv_p2):
        my = lax.axis_index("i")

        barrier_sem = pltpu.get_barrier_semaphore()
        for d in range(1, N_DEV):
            pl.semaphore_signal(barrier_sem, inc=1,
                                device_id=(lax.rem(my + d, N_DEV),),
                                device_id_type=pl.DeviceIdType.MESH)
        pl.semaphore_wait(barrier_sem, N_DEV - 1)

        p1 = []
        for d in range(1, N_DEV):
            t = lax.rem(my + d, N_DEV)
            p1.append(pltpu.make_async_remote_copy(
                src_ref=part_ref.at[pl.ds(t * CH, CH)],
                dst_ref=rs_buf.at[d],
                send_sem=send_p1.at[d],
                recv_sem=recv_p1.at[d],
                device_id=(t,),
                device_id_type=pl.DeviceIdType.MESH,
            ))

        qi = lax.broadcasted_iota(jnp.int32, (SQ, SKV), 0)
        ki = lax.broadcasted_iota(jnp.int32, (SQ, SKV), 1)
        mask = (jnp.abs(qi - ki) <= 128) | (ki < 32) | (qi < 32)
        for b in range(B):
            q_b = jnp.dot(x_ref[b], wq_ref[...],
                          preferred_element_type=jnp.float32)
            ctx_cols = []
            for h in range(H_PER):
                q_h = q_b[:, h * DH:(h + 1) * DH]
                k_h = k_ref[b, :, h, :]
                v_h = v_ref[b, :, h, :]
                s = lax.dot_general(q_h, k_h, (((1,), (1,)), ((), ())),
                                    preferred_element_type=jnp.float32)
                s = s * 0.125
                s = jnp.where(mask, s, -1e9)
                s = s - jnp.max(s, axis=1, keepdims=True)
                w = jnp.exp(s)
                w = w / jnp.sum(w, axis=1, keepdims=True)
                ctx_cols.append(jnp.dot(w, v_h,
                                        preferred_element_type=jnp.float32))
            ctx_b = jnp.concatenate(ctx_cols, axis=1)
            part_b = jnp.dot(ctx_b, wo_ref[...],
                             preferred_element_type=jnp.float32)
            part_ref[b * SQ:(b + 1) * SQ] = part_b.astype(jnp.bfloat16)
            lo, hi = b * (N_DEV // B), (b + 1) * (N_DEV // B)
            for d in range(1, N_DEV):
                t = lax.rem(my + d, N_DEV)

                @pl.when((t >= lo) & (t < hi))
                def _(rdma=p1[d - 1]):
                    rdma.start()

        red = part_ref[pl.ds(my * CH, CH)].astype(jnp.float32)
        for d in range(1, N_DEV):
            p1[d - 1].wait_recv()
            red = red + rs_buf[d].astype(jnp.float32)
        chunk_ref[...] = red.astype(jnp.bfloat16)

        p2 = []
        for d in range(1, N_DEV):
            t = lax.rem(my + d, N_DEV)
            rdma = pltpu.make_async_remote_copy(
                src_ref=chunk_ref,
                dst_ref=acc_ref.at[pl.ds(my * CH, CH)],
                send_sem=send_p2.at[d],
                recv_sem=recv_p2.at[d],
                device_id=(t,),
                device_id_type=pl.DeviceIdType.MESH,
            )
            rdma.start()
            p2.append(rdma)
        acc_ref[pl.ds(my * CH, CH)] = chunk_ref[...]

        for d in range(1, N_DEV):
            p2[d - 1].wait_recv()
        for b in range(B):
            out_ref[b] = acc_ref[b * SQ:(b + 1) * SQ].astype(jnp.float32)

        for d in range(1, N_DEV):
            p1[d - 1].wait_send()
            p2[d - 1].wait_send()

    return pl.pallas_call(
        body,
        out_shape=jax.ShapeDtypeStruct((B, SQ, D_MODEL), jnp.float32),
        in_specs=[pl.BlockSpec(memory_space=pltpu.VMEM)] * 5,
        out_specs=pl.BlockSpec(memory_space=pltpu.VMEM),
        scratch_shapes=[
            pltpu.VMEM((ROWS, D_MODEL), jnp.bfloat16),
            pltpu.VMEM((N_DEV, CH, D_MODEL), jnp.bfloat16),
            pltpu.VMEM((CH, D_MODEL), jnp.bfloat16),
            pltpu.VMEM((ROWS, D_MODEL), jnp.bfloat16),
            pltpu.SemaphoreType.DMA((N_DEV,)),
            pltpu.SemaphoreType.DMA((N_DEV,)),
            pltpu.SemaphoreType.DMA((N_DEV,)),
            pltpu.SemaphoreType.DMA((N_DEV,)),
        ],
        compiler_params=pltpu.CompilerParams(collective_id=0),
    )(x, wq_my, K_ext, V_ext, wo_my)
